# baseline (speedup 1.0000x reference)
import functools

import jax
import jax.numpy as jnp
from jax.experimental import pallas as pl
from jax.experimental.pallas import tpu as pltpu

NUM_EXPERTS = 64
HIDDEN = 4096
TOKENS = 8192
SINKHORN_ITERS = 30
TILE = 1024


def _router_kernel(x_ref, w_ref, b_ref, logits_ref, aff_ref, costT_ref):
    x = x_ref[...].reshape(TILE, HIDDEN)
    logits = jax.lax.dot_general(
        x, w_ref[...], (((1,), (0,)), ((), ())),
        preferred_element_type=jnp.float32,
    ) + b_ref[...]
    logits_ref[...] = logits
    aff_ref[...] = jax.nn.sigmoid(logits)
    costT_ref[...] = jnp.exp(logits).T


def _sinkhorn_kernel(costT_ref, idx_ref):
    costT = costT_ref[...]
    eps = 1e-8

    def body(_, d1):
        d0 = (1.0 / TOKENS) / (jnp.sum(costT * d1, axis=0, keepdims=True) + eps)
        d1 = (1.0 / NUM_EXPERTS) / (jnp.sum(costT * d0, axis=1, keepdims=True) + eps)
        return d1

    d1 = jax.lax.fori_loop(0, SINKHORN_ITERS, body,
                           jnp.ones((NUM_EXPERTS, 1), jnp.float32))
    m = costT * d1
    maxv = jnp.max(m, axis=0, keepdims=True)
    eidx = jax.lax.broadcasted_iota(jnp.int32, (NUM_EXPERTS, TOKENS), 0)
    idx_ref[...] = jnp.min(
        jnp.where(m == maxv, eidx, NUM_EXPERTS), axis=0, keepdims=True)


@functools.partial(jax.jit, static_argnames=())
def kernel(hidden_states, W, b):
    n_tiles = TOKENS // TILE
    logits, aff, costT = pl.pallas_call(
        _router_kernel,
        grid=(n_tiles,),
        in_specs=[
            pl.BlockSpec((TILE // 4, 4, HIDDEN), lambda i: (i, 0, 0)),
            pl.BlockSpec((HIDDEN, NUM_EXPERTS), lambda i: (0, 0)),
            pl.BlockSpec((1, NUM_EXPERTS), lambda i: (0, 0)),
        ],
        out_specs=[
            pl.BlockSpec((TILE, NUM_EXPERTS), lambda i: (i, 0)),
            pl.BlockSpec((TILE, NUM_EXPERTS), lambda i: (i, 0)),
            pl.BlockSpec((NUM_EXPERTS, TILE), lambda i: (0, i)),
        ],
        out_shape=[
            jax.ShapeDtypeStruct((TOKENS, NUM_EXPERTS), jnp.float32),
            jax.ShapeDtypeStruct((TOKENS, NUM_EXPERTS), jnp.float32),
            jax.ShapeDtypeStruct((NUM_EXPERTS, TOKENS), jnp.float32),
        ],
        compiler_params=pltpu.CompilerParams(
            dimension_semantics=("parallel",)),
    )(hidden_states, W, b.reshape(1, NUM_EXPERTS))

    idx = pl.pallas_call(
        _sinkhorn_kernel,
        out_shape=jax.ShapeDtypeStruct((1, TOKENS), jnp.int32),
    )(costT)

    return (logits, aff, idx.reshape(TOKENS, 1))


# speedup vs baseline: 1.0638x; 1.0638x over previous
import functools

import jax
import jax.numpy as jnp
from jax.experimental import pallas as pl
from jax.experimental.pallas import tpu as pltpu

NUM_EXPERTS = 64
HIDDEN = 4096
TOKENS = 8192
SINKHORN_ITERS = 30
TILE = 512


def _router_kernel(x_ref, w_ref, b_ref, logits_ref, aff_ref, idx_ref,
                   costT_ref):
    i = pl.program_id(0)
    x = x_ref[...].reshape(TILE, HIDDEN)
    logits = jax.lax.dot_general(
        x, w_ref[...], (((1,), (0,)), ((), ())),
        preferred_element_type=jnp.float32,
    ) + b_ref[...]
    logits_ref[...] = logits
    aff_ref[...] = jax.nn.sigmoid(logits)
    costT_ref[:, pl.ds(i * TILE, TILE)] = jnp.exp(logits).T

    @pl.when(i == pl.num_programs(0) - 1)
    def _sinkhorn():
        costT = costT_ref[...]
        eps = 1e-8

        def body(_, d1):
            d0 = (1.0 / TOKENS) / (
                jnp.sum(costT * d1, axis=0, keepdims=True) + eps)
            d1 = (1.0 / NUM_EXPERTS) / (
                jnp.sum(costT * d0, axis=1, keepdims=True) + eps)
            return d1

        d1 = jax.lax.fori_loop(0, SINKHORN_ITERS, body,
                               jnp.ones((NUM_EXPERTS, 1), jnp.float32))
        m = costT * d1
        maxv = jnp.max(m, axis=0, keepdims=True)
        eidx = jax.lax.broadcasted_iota(jnp.int32, (NUM_EXPERTS, TOKENS), 0)
        idx_ref[...] = jnp.min(
            jnp.where(m == maxv, eidx, NUM_EXPERTS), axis=0, keepdims=True)


@functools.partial(jax.jit, static_argnames=())
def kernel(hidden_states, W, b):
    n_tiles = TOKENS // TILE
    logits, aff, idx = pl.pallas_call(
        _router_kernel,
        grid=(n_tiles,),
        in_specs=[
            pl.BlockSpec((TILE // 4, 4, HIDDEN), lambda i: (i, 0, 0)),
            pl.BlockSpec((HIDDEN, NUM_EXPERTS), lambda i: (0, 0)),
            pl.BlockSpec((1, NUM_EXPERTS), lambda i: (0, 0)),
        ],
        out_specs=[
            pl.BlockSpec((TILE, NUM_EXPERTS), lambda i: (i, 0)),
            pl.BlockSpec((TILE, NUM_EXPERTS), lambda i: (i, 0)),
            pl.BlockSpec((1, TOKENS), lambda i: (0, 0)),
        ],
        out_shape=[
            jax.ShapeDtypeStruct((TOKENS, NUM_EXPERTS), jnp.float32),
            jax.ShapeDtypeStruct((TOKENS, NUM_EXPERTS), jnp.float32),
            jax.ShapeDtypeStruct((1, TOKENS), jnp.int32),
        ],
        scratch_shapes=[pltpu.VMEM((NUM_EXPERTS, TOKENS), jnp.float32)],
    )(hidden_states, W, b.reshape(1, NUM_EXPERTS))

    return (logits, aff, idx.reshape(TOKENS, 1))
